# Initial kernel scaffold; baseline (speedup 1.0000x reference)
import jax
import jax.numpy as jnp
from jax.experimental import pallas as pl


def kernel(pred, inds):
    raise NotImplementedError("write your pallas kernel here")


# speedup vs baseline: 1.0497x; 1.0497x over previous
import functools

import jax
import jax.numpy as jnp
from jax import lax
from jax.experimental import pallas as pl
from jax.experimental.pallas import tpu as pltpu
from jax.experimental.pallas import tpu_sc as plsc

_PULL_W = 0.25
_PUSH_W = 0.25
_MARGIN = 2.0

_B, _C, _H, _W = 8, 16, 256, 256
_HW = _H * _W
_N = 20
_K = 64

_NC, _NS, _L = 2, 16, 16
_NW = _NC * _NS
_IPW = (_B * _N) // _NW
_TPB = _N // _IPW


def _sc_gather_reduce(pred_flat, inds_flat):
  mesh = plsc.VectorSubcoreMesh(core_axis_name="c", subcore_axis_name="s")

  @functools.partial(
      pl.kernel,
      out_type=[
          jax.ShapeDtypeStruct((_B, _N, _C, _L), jnp.float32),
          jax.ShapeDtypeStruct((_NW, _L), jnp.float32),
      ],
      mesh=mesh,
      scratch_types=[
          pltpu.VMEM((_IPW * _K,), jnp.int32),
          pltpu.VMEM((_IPW * 8, 128), jnp.int32),
          pltpu.VMEM((_IPW * 8, 128), jnp.float32),
          pltpu.VMEM((_IPW, _C, _L), jnp.float32),
          pltpu.VMEM((_L,), jnp.float32),
          pltpu.SemaphoreType.DMA,
      ],
  )
  def k(pred_hbm, inds_hbm, outy_hbm, outsq_hbm, ik_v, idx_v, val_v, y_v,
        sq_v, sem):
    wid = lax.axis_index("s") * _NC + lax.axis_index("c")
    b = wid // _TPB
    base = b * (_C * _HW)

    pltpu.sync_copy(inds_hbm.at[pl.ds(wid * (_IPW * _K), _IPW * _K)], ik_v)

    copies = []
    for j in range(_IPW):
      ikc = [ik_v[pl.ds(j * _K + q * _L, _L)] for q in range(_K // _L)]
      for c in range(_C):
        base_c = base + c * _HW
        for q in range(_K // _L):
          p = c * _K + q * _L
          idx_v[j * 8 + p // 128, pl.ds(p % 128, _L)] = ikc[q] + base_c
      for r in range(j * 8, (j + 1) * 8):
        copies.append(
            pltpu.async_copy(pred_hbm.at[idx_v.at[r]], val_v.at[r], sem))

    sq_acc = jnp.zeros((_L,), jnp.float32)
    for j in range(_IPW):
      for r in range(j * 8, (j + 1) * 8):
        copies[r].wait()
      for c in range(_C):
        y = jnp.zeros((_L,), jnp.float32)
        for q in range(_K // _L):
          p = c * _K + q * _L
          v = val_v[j * 8 + p // 128, pl.ds(p % 128, _L)]
          y = y + v
          sq_acc = sq_acc + v * v
        y_v[j, c, :] = y

    sq_v[...] = sq_acc
    pltpu.sync_copy(y_v, outy_hbm.at[b, pl.ds((wid % _TPB) * _IPW, _IPW)])
    pltpu.sync_copy(sq_v, outsq_hbm.at[wid])

  return k(pred_flat, inds_flat)


def _tc_finish(yvals, sqvals):
  def body(y_ref, sq_ref, o_ref):
    s_ck = jnp.sum(y_ref[...], axis=3)
    a = jnp.sum(sq_ref[...])
    b2 = jnp.sum(s_ck * s_ck)
    pull_all = _PULL_W * (a * (1.0 / _K) - b2 * (1.0 / (_K * _K)))

    s = jnp.sum(s_ck, axis=2) * (1.0 / _K)
    diff = s[:, :, None] - s[:, None, :]
    m = jnp.maximum(_MARGIN - jnp.abs(diff), 0.0)
    eye = (lax.broadcasted_iota(jnp.int32, (_B, _N, _N), 1) ==
           lax.broadcasted_iota(jnp.int32, (_B, _N, _N), 2))
    m = jnp.where(eye, 0.0, m)
    push_all = _PUSH_W * jnp.sum(m) / (_N * (_N - 1))

    col = lax.broadcasted_iota(jnp.int32, (1, 2), 1)
    o_ref[...] = jnp.where(col == 0, pull_all, push_all)

  return pl.pallas_call(
      body,
      out_shape=jax.ShapeDtypeStruct((1, 2), jnp.float32),
  )(yvals, sqvals)


@jax.jit
def kernel(pred, inds):
  pred_flat = pred.reshape(-1)
  inds_flat = inds.reshape(-1)
  yvals, sqvals = _sc_gather_reduce(pred_flat, inds_flat)
  out = _tc_finish(yvals, sqvals)
  return (out[0, 0], out[0, 1])


# speedup vs baseline: 1.7811x; 1.6967x over previous
import functools

import jax
import jax.numpy as jnp
from jax import lax
from jax.experimental import pallas as pl
from jax.experimental.pallas import tpu as pltpu
from jax.experimental.pallas import tpu_sc as plsc

_PULL_W = 0.25
_PUSH_W = 0.25
_MARGIN = 2.0

_B, _C, _H, _W = 8, 16, 256, 256
_HW = _H * _W
_N = 20
_K = 64

_NC, _NS, _L = 2, 16, 16
_NW = _NC * _NS
_CPW = (_B * _C) // _NW
_TPB = _NW // _B
_QR = _H // 4
_NBUF = 6


def _sc_gather_reduce(pred2, inds):
  mesh = plsc.VectorSubcoreMesh(core_axis_name="c", subcore_axis_name="s")

  @functools.partial(
      pl.kernel,
      out_type=[
          jax.ShapeDtypeStruct((_NW, _CPW * _N * _L), jnp.float32),
          jax.ShapeDtypeStruct((_NW, _L), jnp.float32),
      ],
      mesh=mesh,
      compiler_params=pltpu.CompilerParams(needs_layout_passes=False),
      scratch_types=[
          pltpu.VMEM((_N, _K), jnp.int32),
          [pltpu.VMEM((_QR, _W), jnp.float32) for _ in range(_NBUF)],
          pltpu.VMEM((_CPW * _N * _L,), jnp.float32),
          pltpu.VMEM((_L,), jnp.float32),
          pltpu.SemaphoreType.DMA,
      ],
  )
  def k(pred_hbm, inds_hbm, outy_hbm, outsq_hbm, ik_v, bufs, y_v, sq_v, sem):
    wid = lax.axis_index("s") * _NC + lax.axis_index("c")
    b = wid // _TPB
    c0 = (wid % _TPB) * _CPW

    pltpu.sync_copy(inds_hbm.at[b], ik_v)

    copies = {}

    def fire(qi):
      j, qq = divmod(qi, 4)
      rbase = (b * _C + c0 + j) * _H + qq * _QR
      copies[qi] = pltpu.async_copy(
          pred_hbm.at[pl.ds(rbase, _QR)], bufs[qi % _NBUF], sem)

    for qi in range(_NBUF):
      fire(qi)

    sq_acc = jnp.zeros((_L,), jnp.float32)
    for j in range(_CPW):
      for qi in range(4 * j, 4 * j + 4):
        copies.pop(qi).wait()
      img = [bufs[(4 * j + qq) % _NBUF] for qq in range(4)]

      def nbody(n, sq, img=img, j=j):
        y = jnp.zeros((_L,), jnp.float32)
        for q in range(_K // _L):
          p = ik_v[n, pl.ds(q * _L, _L)]
          row = lax.shift_right_logical(p, 8)
          qsel = lax.shift_right_logical(p, 14)
          rowm = lax.bitwise_and(row, _QR - 1)
          col = lax.bitwise_and(p, _W - 1)
          g = plsc.load_gather(img[0], [rowm, col])
          for qq in range(1, 4):
            gq = plsc.load_gather(img[qq], [rowm, col])
            g = jnp.where(qsel == qq, gq, g)
          y = y + g
          sq = sq + g * g
        y_v[pl.ds(j * (_N * _L) + n * _L, _L)] = y
        return sq

      sq_acc = lax.fori_loop(0, _N, nbody, sq_acc)
      for qi in range(4 * j + _NBUF, min(4 * j + _NBUF + 4, 4 * _CPW)):
        fire(qi)

    sq_v[...] = sq_acc
    pltpu.sync_copy(y_v, outy_hbm.at[wid])
    pltpu.sync_copy(sq_v, outsq_hbm.at[wid])

  return k(pred2, inds)


def _tc_finish(yvals, sqvals):
  def body(y_ref, sq_ref, o_ref):
    x = y_ref[...]
    gid = lax.broadcasted_iota(jnp.int32, (_CPW * _N * _L, _CPW * _N), 0)
    gcol = lax.broadcasted_iota(jnp.int32, (_CPW * _N * _L, _CPW * _N), 1)
    m1 = (gid // _L == gcol).astype(jnp.float32)
    s_ck = jnp.dot(x, m1, preferred_element_type=jnp.float32)

    a = jnp.sum(sq_ref[...])
    b2 = jnp.sum(s_ck * s_ck)
    pull_all = _PULL_W * (a * (1.0 / _K) - b2 * (1.0 / (_K * _K)))

    e_b = lax.broadcasted_iota(jnp.int32, (_B, _NW), 0)
    e_t = lax.broadcasted_iota(jnp.int32, (_B, _NW), 1)
    e = (e_t // _TPB == e_b).astype(jnp.float32)
    f_c = lax.broadcasted_iota(jnp.int32, (_CPW * _N, _N), 0)
    f_n = lax.broadcasted_iota(jnp.int32, (_CPW * _N, _N), 1)
    f = (f_c % _N == f_n).astype(jnp.float32)
    s = jnp.dot(jnp.dot(e, s_ck, preferred_element_type=jnp.float32), f,
                preferred_element_type=jnp.float32) * (1.0 / _K)

    diff = s[:, :, None] - s[:, None, :]
    m = jnp.maximum(_MARGIN - jnp.abs(diff), 0.0)
    eye = (lax.broadcasted_iota(jnp.int32, (_B, _N, _N), 1) ==
           lax.broadcasted_iota(jnp.int32, (_B, _N, _N), 2))
    m = jnp.where(eye, 0.0, m)
    push_all = _PUSH_W * jnp.sum(m) / (_N * (_N - 1))

    col = lax.broadcasted_iota(jnp.int32, (1, 2), 1)
    o_ref[...] = jnp.where(col == 0, pull_all, push_all)

  return pl.pallas_call(
      body,
      out_shape=jax.ShapeDtypeStruct((1, 2), jnp.float32),
  )(yvals, sqvals)


@jax.jit
def kernel(pred, inds):
  pred2 = pred.reshape(_B * _C * _H, _W)
  yvals, sqvals = _sc_gather_reduce(pred2, inds)
  out = _tc_finish(yvals, sqvals)
  return (out[0, 0], out[0, 1])


# speedup vs baseline: 1.8456x; 1.0362x over previous
import functools

import jax
import jax.numpy as jnp
from jax import lax
from jax.experimental import pallas as pl
from jax.experimental.pallas import tpu as pltpu
from jax.experimental.pallas import tpu_sc as plsc

_PULL_W = 0.25
_PUSH_W = 0.25
_MARGIN = 2.0

_B, _C, _H, _W = 8, 16, 256, 256
_HW = _H * _W
_N = 20
_K = 64

_NC, _NS, _L = 2, 16, 16
_NW = _NC * _NS
_CPW = (_B * _C) // _NW
_TPB = _NW // _B
_QR = _H // 4
_NBUF = 6


def _sc_gather_reduce(pred2, inds):
  mesh = plsc.VectorSubcoreMesh(core_axis_name="c", subcore_axis_name="s")

  @functools.partial(
      pl.kernel,
      out_type=[
          jax.ShapeDtypeStruct((_NW, _CPW * _N * _L), jnp.float32),
          jax.ShapeDtypeStruct((_NW, _L), jnp.float32),
      ],
      mesh=mesh,
      compiler_params=pltpu.CompilerParams(needs_layout_passes=False),
      scratch_types=[
          pltpu.VMEM((_N, _K), jnp.int32),
          [pltpu.VMEM((_QR, _W), jnp.float32) for _ in range(_NBUF)],
          pltpu.VMEM((_CPW * _N * _L,), jnp.float32),
          pltpu.VMEM((_L,), jnp.float32),
          pltpu.SemaphoreType.DMA,
      ],
  )
  def k(pred_hbm, inds_hbm, outy_hbm, outsq_hbm, ik_v, bufs, y_v, sq_v, sem):
    wid = lax.axis_index("s") * _NC + lax.axis_index("c")
    b = wid // _TPB
    c0 = (wid % _TPB) * _CPW

    pltpu.sync_copy(inds_hbm.at[b], ik_v)

    copies = {}

    def fire(qi):
      j, qq = divmod(qi, 4)
      rbase = (b * _C + c0 + j) * _H + qq * _QR
      copies[qi] = pltpu.async_copy(
          pred_hbm.at[pl.ds(rbase, _QR)], bufs[qi % _NBUF], sem)

    for qi in range(_NBUF):
      fire(qi)

    sq_acc = jnp.zeros((_L,), jnp.float32)
    for j in range(_CPW):
      for half in range(2):
        q0 = 4 * j + 2 * half
        copies.pop(q0).wait()
        copies.pop(q0 + 1).wait()
        blo, bhi = bufs[q0 % _NBUF], bufs[(q0 + 1) % _NBUF]

        def nbody(n, sq, blo=blo, bhi=bhi, half=half, j=j):
          y = jnp.zeros((_L,), jnp.float32)
          for q in range(_K // _L):
            p = ik_v[n, pl.ds(q * _L, _L)]
            row = lax.shift_right_logical(p, 8)
            qsel = lax.bitwise_and(lax.shift_right_logical(p, 14), 1)
            hsel = lax.shift_right_logical(p, 15)
            rowm = lax.bitwise_and(row, _QR - 1)
            col = lax.bitwise_and(p, _W - 1)
            glo = plsc.load_gather(blo, [rowm, col])
            ghi = plsc.load_gather(bhi, [rowm, col])
            g = jnp.where(qsel == 0, glo, ghi)
            v = jnp.where(hsel == half, g, 0.0)
            y = y + v
            sq = sq + v * v
          o = j * (_N * _L) + n * _L
          if half == 0:
            y_v[pl.ds(o, _L)] = y
          else:
            y_v[pl.ds(o, _L)] = y_v[pl.ds(o, _L)] + y
          return sq

        sq_acc = lax.fori_loop(0, _N, nbody, sq_acc)
        for qi in range(q0 + _NBUF, min(q0 + _NBUF + 2, 4 * _CPW)):
          fire(qi)

    sq_v[...] = sq_acc
    pltpu.sync_copy(y_v, outy_hbm.at[wid])
    pltpu.sync_copy(sq_v, outsq_hbm.at[wid])

  return k(pred2, inds)


def _tc_finish(yvals, sqvals):
  def body(y_ref, sq_ref, o_ref, o2_ref):
    x = y_ref[...]
    gid = lax.broadcasted_iota(jnp.int32, (_CPW * _N * _L, _CPW * _N), 0)
    gcol = lax.broadcasted_iota(jnp.int32, (_CPW * _N * _L, _CPW * _N), 1)
    m1 = (gid // _L == gcol).astype(jnp.float32)
    s_ck = jnp.dot(x, m1, preferred_element_type=jnp.float32)

    a = jnp.sum(sq_ref[...])
    b2 = jnp.sum(s_ck * s_ck)
    pull_all = _PULL_W * (a * (1.0 / _K) - b2 * (1.0 / (_K * _K)))

    e_b = lax.broadcasted_iota(jnp.int32, (_B, _NW), 0)
    e_t = lax.broadcasted_iota(jnp.int32, (_B, _NW), 1)
    e = (e_t // _TPB == e_b).astype(jnp.float32)
    f_c = lax.broadcasted_iota(jnp.int32, (_CPW * _N, _N), 0)
    f_n = lax.broadcasted_iota(jnp.int32, (_CPW * _N, _N), 1)
    f = (f_c % _N == f_n).astype(jnp.float32)
    s = jnp.dot(jnp.dot(e, s_ck, preferred_element_type=jnp.float32), f,
                preferred_element_type=jnp.float32) * (1.0 / _K)

    diff = s[:, :, None] - s[:, None, :]
    m = jnp.maximum(_MARGIN - jnp.abs(diff), 0.0)
    eye = (lax.broadcasted_iota(jnp.int32, (_B, _N, _N), 1) ==
           lax.broadcasted_iota(jnp.int32, (_B, _N, _N), 2))
    m = jnp.where(eye, 0.0, m)
    push_all = _PUSH_W * jnp.sum(m) / (_N * (_N - 1))

    o_ref[...] = jnp.zeros((1, 1), jnp.float32) + pull_all
    o2_ref[...] = jnp.zeros((1, 1), jnp.float32) + push_all

  return pl.pallas_call(
      body,
      out_shape=[
          jax.ShapeDtypeStruct((1, 1), jnp.float32),
          jax.ShapeDtypeStruct((1, 1), jnp.float32),
      ],
  )(yvals, sqvals)


@jax.jit
def kernel(pred, inds):
  pred2 = pred.reshape(_B * _C * _H, _W)
  yvals, sqvals = _sc_gather_reduce(pred2, inds)
  pull_all, push_all = _tc_finish(yvals, sqvals)
  return (pull_all.reshape(()), push_all.reshape(()))
